# baseline (speedup 1.0000x reference)
import functools

import jax
import jax.numpy as jnp
from jax import lax
from jax.experimental import pallas as pl
from jax.experimental.pallas import tpu as pltpu
from jax.experimental.pallas import tpu_sc as plsc

N_NODES = 10000
N_EDGES = 160000
CHANNELS = 256
QUART = CHANNELS // 4
NSUB = 16
NBUF = 5
CHUNK = 80
CPS = 125
NOUTER = CPS // NBUF
NCHUNK_TOT = N_EDGES // CHUNK
ROWS_PER_SUB = N_NODES // NSUB


def _sc_agg(x, s2, t2):
  mesh = plsc.VectorSubcoreMesh(core_axis_name="c", subcore_axis_name="s")

  @functools.partial(
      pl.kernel,
      out_type=jax.ShapeDtypeStruct((N_NODES, CHANNELS), jnp.float32),
      mesh=mesh,
      scratch_types=[
          pltpu.VMEM((CPS, CHUNK), jnp.int32),
          pltpu.VMEM((CPS, CHUNK), jnp.int32),
          pltpu.VMEM((NBUF, CHUNK, QUART), jnp.float32),
          pltpu.VMEM_SHARED((N_NODES, QUART), jnp.float32),
          pltpu.VMEM_SHARED((N_NODES, QUART), jnp.float32),
          pltpu.SemaphoreType.DMA,
          pltpu.SemaphoreType.DMA,
          pltpu.SemaphoreType.DMA,
          pltpu.SemaphoreType.DMA,
          pltpu.SemaphoreType.DMA,
          pltpu.SemaphoreType.DMA,
          pltpu.SemaphoreType.DMA,
          pltpu.SemaphoreType.DMA,
          pltpu.SemaphoreType.DMA,
          pltpu.SemaphoreType.DMA,
      ],
      compiler_params=pltpu.CompilerParams(use_tc_tiling_on_sc=False),
  )
  def k(x_hbm, s_hbm, t_hbm, out_hbm, sidx, tidx, rows, table, slab,
        g0, g1, g2, g3, g4, s0, s1, s2b, s3, s4):
    gsem = [g0, g1, g2, g3, g4]
    ssem = [s0, s1, s2b, s3, s4]
    c = lax.axis_index("c")
    s = lax.axis_index("s")
    nsl = pl.ds(s * ROWS_PER_SUB, ROWS_PER_SUB)
    cbase = s * CPS

    pltpu.sync_copy(s_hbm.at[pl.ds(cbase, CPS)], sidx)
    pltpu.sync_copy(t_hbm.at[pl.ds(cbase, CPS)], tidx)

    for p in range(2):
      q = c * 2 + p
      csl = pl.ds(q * QUART, QUART)

      pltpu.sync_copy(x_hbm.at[nsl, csl], table.at[nsl])
      pltpu.sync_copy(x_hbm.at[nsl, csl], slab.at[nsl])
      plsc.subcore_barrier()

      def outer(g):
        for b in range(NBUF):
          @pl.when(g > 0)
          def _drain_prev(b=b):
            pltpu.make_async_copy(
                rows.at[b], slab.at[tidx.at[(g - 1) * NBUF + b]],
                ssem[b]).wait()
          pltpu.async_copy(table.at[sidx.at[g * NBUF + b]], rows.at[b],
                           gsem[b])
        for b in range(NBUF):
          pltpu.make_async_copy(table.at[sidx.at[g * NBUF + b]], rows.at[b],
                                gsem[b]).wait()
          pltpu.async_copy(rows.at[b], slab.at[tidx.at[g * NBUF + b]],
                           ssem[b], add=True)

      pl.loop(0, NOUTER)(outer)
      for b in range(NBUF):
        pltpu.make_async_copy(
            rows.at[b], slab.at[tidx.at[(NOUTER - 1) * NBUF + b]],
            ssem[b]).wait()
      plsc.subcore_barrier()

      pltpu.sync_copy(slab.at[nsl], out_hbm.at[nsl, csl])
      plsc.subcore_barrier()

  return k(x, s2, t2)


def _mm_body(agg_ref, norm_ref, w_ref, out_ref):
  h = norm_ref[...] * agg_ref[...]
  out_ref[...] = jnp.dot(h, w_ref[...], preferred_element_type=jnp.float32)


def _tc_matmul(agg, norm, weight):
  bn = 512
  grid = (pl.cdiv(N_NODES, bn),)
  return pl.pallas_call(
      _mm_body,
      grid=grid,
      in_specs=[
          pl.BlockSpec((bn, CHANNELS), lambda i: (i, 0)),
          pl.BlockSpec((bn, 1), lambda i: (i, 0)),
          pl.BlockSpec((CHANNELS, CHANNELS), lambda i: (0, 0)),
      ],
      out_specs=pl.BlockSpec((bn, CHANNELS), lambda i: (i, 0)),
      out_shape=jax.ShapeDtypeStruct((N_NODES, CHANNELS), jnp.float32),
  )(agg, norm, weight)


def kernel(x, sources, targets, norm, weight):
  s2 = sources.astype(jnp.int32).reshape(NCHUNK_TOT, CHUNK)
  t2 = targets.astype(jnp.int32).reshape(NCHUNK_TOT, CHUNK)
  agg = _sc_agg(x, s2, t2)
  return _tc_matmul(agg, norm, weight)


# speedup vs baseline: 1.2082x; 1.2082x over previous
import functools

import jax
import jax.numpy as jnp
from jax import lax
from jax.experimental import pallas as pl
from jax.experimental.pallas import tpu as pltpu
from jax.experimental.pallas import tpu_sc as plsc

N_NODES = 10000
N_EDGES = 160000
CHANNELS = 256
QUART = CHANNELS // 4
NSUB = 16
NBUF = 5
CHUNK = 80
CPS = 125
NOUTER = CPS // NBUF
NCHUNK_TOT = N_EDGES // CHUNK
ROWS_PER_SUB = N_NODES // NSUB


def _sc_agg(x, s2, t2):
  mesh = plsc.VectorSubcoreMesh(core_axis_name="c", subcore_axis_name="s")

  @functools.partial(
      pl.kernel,
      out_type=jax.ShapeDtypeStruct((N_NODES, CHANNELS), jnp.float32),
      mesh=mesh,
      scratch_types=[
          pltpu.VMEM((CPS, CHUNK), jnp.int32),
          pltpu.VMEM((CPS, CHUNK), jnp.int32),
          pltpu.VMEM((NBUF, CHUNK, QUART), jnp.float32),
          pltpu.VMEM_SHARED((N_NODES, QUART), jnp.float32),
          pltpu.VMEM_SHARED((N_NODES, QUART), jnp.float32),
          pltpu.SemaphoreType.DMA,
          pltpu.SemaphoreType.DMA,
          pltpu.SemaphoreType.DMA,
          pltpu.SemaphoreType.DMA,
          pltpu.SemaphoreType.DMA,
      ],
      compiler_params=pltpu.CompilerParams(use_tc_tiling_on_sc=False),
  )
  def k(x_hbm, s_hbm, t_hbm, out_hbm, sidx, tidx, rows, table, slab,
        g0, g1, g2, g3, g4):
    gsem = [g0, g1, g2, g3, g4]
    c = lax.axis_index("c")
    s = lax.axis_index("s")
    nsl = pl.ds(s * ROWS_PER_SUB, ROWS_PER_SUB)
    cbase = s * CPS

    pltpu.sync_copy(s_hbm.at[pl.ds(cbase, CPS)], sidx)
    pltpu.sync_copy(t_hbm.at[pl.ds(cbase, CPS)], tidx)

    for p in range(2):
      q = c * 2 + p
      csl = pl.ds(q * QUART, QUART)

      pltpu.sync_copy(x_hbm.at[nsl, csl], table.at[nsl])
      pltpu.sync_copy(x_hbm.at[nsl, csl], slab.at[nsl])
      plsc.subcore_barrier()

      def outer(g):
        for b in range(NBUF):
          pltpu.async_copy(table.at[sidx.at[g * NBUF + b]], rows.at[b],
                           gsem[b])
        for b in range(NBUF):
          pltpu.make_async_copy(table.at[sidx.at[g * NBUF + b]], rows.at[b],
                                gsem[b]).wait()
          pltpu.sync_copy(rows.at[b], slab.at[tidx.at[g * NBUF + b]],
                          add=True)

      pl.loop(0, NOUTER)(outer)
      plsc.subcore_barrier()

      pltpu.sync_copy(slab.at[nsl], out_hbm.at[nsl, csl])
      plsc.subcore_barrier()

  return k(x, s2, t2)


def _mm_body(agg_ref, norm_ref, w_ref, out_ref):
  h = norm_ref[...] * agg_ref[...]
  out_ref[...] = jnp.dot(h, w_ref[...], preferred_element_type=jnp.float32)


def _tc_matmul(agg, norm, weight):
  bn = 512
  grid = (pl.cdiv(N_NODES, bn),)
  return pl.pallas_call(
      _mm_body,
      grid=grid,
      in_specs=[
          pl.BlockSpec((bn, CHANNELS), lambda i: (i, 0)),
          pl.BlockSpec((bn, 1), lambda i: (i, 0)),
          pl.BlockSpec((CHANNELS, CHANNELS), lambda i: (0, 0)),
      ],
      out_specs=pl.BlockSpec((bn, CHANNELS), lambda i: (i, 0)),
      out_shape=jax.ShapeDtypeStruct((N_NODES, CHANNELS), jnp.float32),
  )(agg, norm, weight)


def kernel(x, sources, targets, norm, weight):
  s2 = sources.astype(jnp.int32).reshape(NCHUNK_TOT, CHUNK)
  t2 = targets.astype(jnp.int32).reshape(NCHUNK_TOT, CHUNK)
  agg = _sc_agg(x, s2, t2)
  return _tc_matmul(agg, norm, weight)
